# Initial kernel scaffold; baseline (speedup 1.0000x reference)
import jax
import jax.numpy as jnp
from jax.experimental import pallas as pl


def kernel(inputs, edge_index, W, b):
    raise NotImplementedError("write your pallas kernel here")


# speedup vs baseline: 8.8115x; 8.8115x over previous
import functools

import jax
import jax.numpy as jnp
from jax import lax
from jax.experimental import pallas as pl
from jax.experimental.pallas import tpu as pltpu
from jax.experimental.pallas import tpu_sc as plsc

N = 10000
E = 320000
D = 128
NC, NS, L = 2, 16, 16
NW = NC * NS
EPW = E // NW
DEG_R = EPW // L
CH = 100
NCH = EPW // CH
NG = 5
GCH = NCH // NG
ZR = 8
NZCH = N // ZR
WR = 80
NWCH = N // WR

_mesh = plsc.VectorSubcoreMesh(
    core_axis_name="c", subcore_axis_name="s", num_cores=NC, num_subcores=NS
)


@functools.partial(
    pl.kernel,
    out_type=jax.ShapeDtypeStruct((NW, N), jnp.float32),
    mesh=_mesh,
    scratch_types=[
        pltpu.VMEM((DEG_R, L), jnp.int32),
        pltpu.VMEM((N,), jnp.float32),
    ],
    compiler_params=pltpu.CompilerParams(needs_layout_passes=False),
)
def _degree_kernel(dst_hbm, deg_out, dst_v, deg_v):
    wid = lax.axis_index("s") * NC + lax.axis_index("c")
    pltpu.sync_copy(dst_hbm.at[wid], dst_v)

    zeros = jnp.zeros((L,), jnp.float32)

    def zrow(i, carry):
        deg_v[pl.ds(i * L, L)] = zeros
        return carry

    lax.fori_loop(0, N // L, zrow, 0)

    ones = jnp.ones((L,), jnp.float32)

    def body(i, carry):
        idx = dst_v[i, :]
        plsc.addupdate_scatter(deg_v, [idx], ones)
        return carry

    lax.fori_loop(0, DEG_R, body, 0)
    pltpu.sync_copy(deg_v, deg_out.at[wid])


@functools.partial(
    pl.kernel,
    out_type=jax.ShapeDtypeStruct((NC, N, D), jnp.float32),
    mesh=_mesh,
    scratch_types=[
        pltpu.VMEM((GCH, CH), jnp.int32),
        pltpu.VMEM((GCH, CH), jnp.int32),
        pltpu.VMEM((2, CH, D), jnp.float32),
        pltpu.VMEM((ZR, D), jnp.float32),
        pltpu.VMEM_SHARED((N, D), jnp.float32),
        pltpu.SemaphoreType.DMA,
        pltpu.SemaphoreType.DMA,
    ],
    compiler_params=pltpu.CompilerParams(needs_layout_passes=False),
)
def _agg_kernel(z_hbm, src_hbm, dst_hbm, agg_out,
                src_v, dst_v, rows_v, zb_v, acc_sh, sem0, sem1):
    cid = lax.axis_index("c")
    sid = lax.axis_index("s")
    wid = sid * NC + cid

    zeros = jnp.zeros((L,), jnp.float32)

    def zbody(k, carry):
        r = k // (D // L)
        c = (k % (D // L)) * L
        zb_v[r, pl.ds(c, L)] = zeros
        return carry

    lax.fori_loop(0, ZR * (D // L), zbody, 0)

    def zstripe(k, carry):
        cidx = sid + NS * k

        @pl.when(cidx < NZCH)
        def _():
            off = pl.multiple_of(cidx * ZR, 8)
            pltpu.sync_copy(zb_v, acc_sh.at[pl.ds(off, ZR)])

        return carry

    lax.fori_loop(0, (NZCH + NS - 1) // NS, zstripe, 0)
    plsc.subcore_barrier()

    def group(g, carry):
        pltpu.sync_copy(src_hbm.at[wid, g], src_v)
        pltpu.sync_copy(dst_hbm.at[wid, g], dst_v)
        pltpu.async_copy(z_hbm.at[src_v.at[0]], rows_v.at[0], sem0)

        def pair(i, carry2):
            j0 = 2 * i
            pltpu.make_async_copy(
                z_hbm.at[src_v.at[j0]], rows_v.at[0], sem0).wait()
            pltpu.async_copy(z_hbm.at[src_v.at[j0 + 1]], rows_v.at[1], sem1)
            pltpu.sync_copy(rows_v.at[0], acc_sh.at[dst_v.at[j0]], add=True)

            pltpu.make_async_copy(
                z_hbm.at[src_v.at[j0 + 1]], rows_v.at[1], sem1).wait()

            @pl.when(j0 + 2 < GCH)
            def _():
                pltpu.async_copy(
                    z_hbm.at[src_v.at[j0 + 2]], rows_v.at[0], sem0)

            pltpu.sync_copy(
                rows_v.at[1], acc_sh.at[dst_v.at[j0 + 1]], add=True)
            return carry2

        lax.fori_loop(0, GCH // 2, pair, 0)
        return carry

    lax.fori_loop(0, NG, group, 0)

    plsc.subcore_barrier()

    def wstripe(k, carry):
        cidx = sid + NS * k

        @pl.when(cidx < NWCH)
        def _():
            off = pl.multiple_of(cidx * WR, 8)
            pltpu.sync_copy(
                acc_sh.at[pl.ds(off, WR)],
                agg_out.at[cid].at[pl.ds(off, WR)],
            )

        return carry

    lax.fori_loop(0, (NWCH + NS - 1) // NS, wstripe, 0)


BLK = 2000


def _prep_body(x_ref, degp_ref, w0t_ref, b_ref, z_ref, p_ref, norm_ref):
    deg = jnp.sum(degp_ref[...], axis=1, keepdims=True)
    norm = lax.rsqrt(jnp.maximum(deg, 1.0))
    x = x_ref[...]
    z_ref[...] = x * norm
    norm_ref[...] = norm
    p_ref[...] = (
        jnp.dot(x, w0t_ref[...], preferred_element_type=jnp.float32)
        + b_ref[...]
    )


def _prep(x, degp_t, w0t, b2):
    grid = (N // BLK,)
    return pl.pallas_call(
        _prep_body,
        grid=grid,
        in_specs=[
            pl.BlockSpec((BLK, D), lambda i: (i, 0)),
            pl.BlockSpec((BLK, NW), lambda i: (i, 0)),
            pl.BlockSpec((D, D), lambda i: (0, 0)),
            pl.BlockSpec((1, D), lambda i: (0, 0)),
        ],
        out_specs=[
            pl.BlockSpec((BLK, D), lambda i: (i, 0)),
            pl.BlockSpec((BLK, D), lambda i: (i, 0)),
            pl.BlockSpec((BLK, 1), lambda i: (i, 0)),
        ],
        out_shape=[
            jax.ShapeDtypeStruct((N, D), jnp.float32),
            jax.ShapeDtypeStruct((N, D), jnp.float32),
            jax.ShapeDtypeStruct((N, 1), jnp.float32),
        ],
    )(x, degp_t, w0t, b2)


def _final_body(p_ref, a0_ref, a1_ref, norm_ref, w1t_ref, out_ref):
    a = (a0_ref[...] + a1_ref[...]) * norm_ref[...]
    out_ref[...] = p_ref[...] - jnp.dot(
        a, w1t_ref[...], preferred_element_type=jnp.float32
    )


def _final(p, a0, a1, norm, w1t):
    grid = (N // BLK,)
    return pl.pallas_call(
        _final_body,
        grid=grid,
        in_specs=[
            pl.BlockSpec((BLK, D), lambda i: (i, 0)),
            pl.BlockSpec((BLK, D), lambda i: (i, 0)),
            pl.BlockSpec((BLK, D), lambda i: (i, 0)),
            pl.BlockSpec((BLK, 1), lambda i: (i, 0)),
            pl.BlockSpec((D, D), lambda i: (0, 0)),
        ],
        out_specs=pl.BlockSpec((BLK, D), lambda i: (i, 0)),
        out_shape=jax.ShapeDtypeStruct((N, D), jnp.float32),
    )(p, a0, a1, norm, w1t)


def kernel(inputs, edge_index, W, b):
    src = edge_index[0].reshape(NW, NG, GCH, CH)
    dst = edge_index[1].reshape(NW, NG, GCH, CH)
    dst16 = edge_index[1].reshape(NW, DEG_R, L)

    degp = _degree_kernel(dst16)
    degp_t = degp.T

    w0t = W[:, :D].T
    w1t = W[:, D:].T
    z, p, norm = _prep(inputs, degp_t, w0t, b.reshape(1, D))

    agg = _agg_kernel(z, src, dst)
    out = _final(p, agg[0], agg[1], norm, w1t)
    return out


# speedup vs baseline: 9.7770x; 1.1096x over previous
import functools

import jax
import jax.numpy as jnp
from jax import lax
from jax.experimental import pallas as pl
from jax.experimental.pallas import tpu as pltpu
from jax.experimental.pallas import tpu_sc as plsc

N = 10000
E = 320000
D = 128
NC, NS, L = 2, 16, 16
NW = NC * NS
EPW = E // NW
DEG_R = EPW // L
CH = 50
NCH = EPW // CH
NG = 5
GCH = NCH // NG
RB = 4
ZR = 8
NZCH = N // ZR
WR = 80
NWCH = N // WR

_mesh = plsc.VectorSubcoreMesh(
    core_axis_name="c", subcore_axis_name="s", num_cores=NC, num_subcores=NS
)


@functools.partial(
    pl.kernel,
    out_type=jax.ShapeDtypeStruct((NW, N), jnp.float32),
    mesh=_mesh,
    scratch_types=[
        pltpu.VMEM((DEG_R, L), jnp.int32),
        pltpu.VMEM((N,), jnp.float32),
    ],
    compiler_params=pltpu.CompilerParams(needs_layout_passes=False),
)
def _degree_kernel(dst_hbm, deg_out, dst_v, deg_v):
    wid = lax.axis_index("s") * NC + lax.axis_index("c")
    pltpu.sync_copy(dst_hbm.at[wid], dst_v)

    zeros = jnp.zeros((L,), jnp.float32)

    def zrow(i, carry):
        deg_v[pl.ds(i * L, L)] = zeros
        return carry

    lax.fori_loop(0, N // L, zrow, 0)

    ones = jnp.ones((L,), jnp.float32)

    def body(i, carry):
        idx = dst_v[i, :]
        plsc.addupdate_scatter(deg_v, [idx], ones)
        return carry

    lax.fori_loop(0, DEG_R, body, 0)
    pltpu.sync_copy(deg_v, deg_out.at[wid])


@functools.partial(
    pl.kernel,
    out_type=jax.ShapeDtypeStruct((NC, N, D), jnp.float32),
    mesh=_mesh,
    scratch_types=[
        pltpu.VMEM((GCH, CH), jnp.int32),
        pltpu.VMEM((GCH, CH), jnp.int32),
        pltpu.VMEM((RB, CH, D), jnp.float32),
        pltpu.VMEM((ZR, D), jnp.float32),
        pltpu.VMEM_SHARED((N, D), jnp.float32),
        [pltpu.SemaphoreType.DMA] * RB,
        [pltpu.SemaphoreType.DMA] * RB,
    ],
    compiler_params=pltpu.CompilerParams(needs_layout_passes=False),
)
def _agg_kernel(z_hbm, src_hbm, dst_hbm, agg_out,
                src_v, dst_v, rows_v, zb_v, acc_sh, gsems, ssems):
    cid = lax.axis_index("c")
    sid = lax.axis_index("s")
    wid = sid * NC + cid

    zeros = jnp.zeros((L,), jnp.float32)

    def zbody(k, carry):
        r = k // (D // L)
        c = (k % (D // L)) * L
        zb_v[r, pl.ds(c, L)] = zeros
        return carry

    lax.fori_loop(0, ZR * (D // L), zbody, 0)

    def zstripe(k, carry):
        cidx = sid + NS * k

        @pl.when(cidx < NZCH)
        def _():
            off = pl.multiple_of(cidx * ZR, 8)
            pltpu.sync_copy(zb_v, acc_sh.at[pl.ds(off, ZR)])

        return carry

    lax.fori_loop(0, (NZCH + NS - 1) // NS, zstripe, 0)
    plsc.subcore_barrier()

    def group(g, carry):
        pltpu.sync_copy(src_hbm.at[wid, g], src_v)
        pltpu.sync_copy(dst_hbm.at[wid, g], dst_v)
        for b in range(RB):
            pltpu.async_copy(z_hbm.at[src_v.at[b]], rows_v.at[b], gsems[b])

        def ring(r, carry2):
            j0 = RB * r
            descs = []
            for b in range(RB):
                j = j0 + b
                pltpu.make_async_copy(
                    z_hbm.at[src_v.at[j]], rows_v.at[b], gsems[b]).wait()
                descs.append(pltpu.async_copy(
                    rows_v.at[b], acc_sh.at[dst_v.at[j]], ssems[b], add=True))
            for b in range(RB):
                j = j0 + b
                descs[b].wait()

                @pl.when(j + RB < GCH)
                def _():
                    pltpu.async_copy(
                        z_hbm.at[src_v.at[j + RB]], rows_v.at[b], gsems[b])

            return carry2

        lax.fori_loop(0, GCH // RB, ring, 0)
        return carry

    lax.fori_loop(0, NG, group, 0)

    plsc.subcore_barrier()

    def wstripe(k, carry):
        cidx = sid + NS * k

        @pl.when(cidx < NWCH)
        def _():
            off = pl.multiple_of(cidx * WR, 8)
            pltpu.sync_copy(
                acc_sh.at[pl.ds(off, WR)],
                agg_out.at[cid].at[pl.ds(off, WR)],
            )

        return carry

    lax.fori_loop(0, (NWCH + NS - 1) // NS, wstripe, 0)


BLK = 2000


def _prep_body(x_ref, degp_ref, w0t_ref, b_ref, z_ref, p_ref, norm_ref):
    deg = jnp.sum(degp_ref[...], axis=1, keepdims=True)
    norm = lax.rsqrt(jnp.maximum(deg, 1.0))
    x = x_ref[...]
    z_ref[...] = x * norm
    norm_ref[...] = norm
    p_ref[...] = (
        jnp.dot(x, w0t_ref[...], preferred_element_type=jnp.float32)
        + b_ref[...]
    )


def _prep(x, degp_t, w0t, b2):
    grid = (N // BLK,)
    return pl.pallas_call(
        _prep_body,
        grid=grid,
        in_specs=[
            pl.BlockSpec((BLK, D), lambda i: (i, 0)),
            pl.BlockSpec((BLK, NW), lambda i: (i, 0)),
            pl.BlockSpec((D, D), lambda i: (0, 0)),
            pl.BlockSpec((1, D), lambda i: (0, 0)),
        ],
        out_specs=[
            pl.BlockSpec((BLK, D), lambda i: (i, 0)),
            pl.BlockSpec((BLK, D), lambda i: (i, 0)),
            pl.BlockSpec((BLK, 1), lambda i: (i, 0)),
        ],
        out_shape=[
            jax.ShapeDtypeStruct((N, D), jnp.float32),
            jax.ShapeDtypeStruct((N, D), jnp.float32),
            jax.ShapeDtypeStruct((N, 1), jnp.float32),
        ],
    )(x, degp_t, w0t, b2)


def _final_body(p_ref, a0_ref, a1_ref, norm_ref, w1t_ref, out_ref):
    a = (a0_ref[...] + a1_ref[...]) * norm_ref[...]
    out_ref[...] = p_ref[...] - jnp.dot(
        a, w1t_ref[...], preferred_element_type=jnp.float32
    )


def _final(p, a0, a1, norm, w1t):
    grid = (N // BLK,)
    return pl.pallas_call(
        _final_body,
        grid=grid,
        in_specs=[
            pl.BlockSpec((BLK, D), lambda i: (i, 0)),
            pl.BlockSpec((BLK, D), lambda i: (i, 0)),
            pl.BlockSpec((BLK, D), lambda i: (i, 0)),
            pl.BlockSpec((BLK, 1), lambda i: (i, 0)),
            pl.BlockSpec((D, D), lambda i: (0, 0)),
        ],
        out_specs=pl.BlockSpec((BLK, D), lambda i: (i, 0)),
        out_shape=jax.ShapeDtypeStruct((N, D), jnp.float32),
    )(p, a0, a1, norm, w1t)


def kernel(inputs, edge_index, W, b):
    src = edge_index[0].reshape(NW, NG, GCH, CH)
    dst = edge_index[1].reshape(NW, NG, GCH, CH)
    dst16 = edge_index[1].reshape(NW, DEG_R, L)

    degp = _degree_kernel(dst16)
    degp_t = degp.T

    w0t = W[:, :D].T
    w1t = W[:, D:].T
    z, p, norm = _prep(inputs, degp_t, w0t, b.reshape(1, D))

    agg = _agg_kernel(z, src, dst)
    out = _final(p, agg[0], agg[1], norm, w1t)
    return out


# speedup vs baseline: 10.1220x; 1.0353x over previous
import functools

import jax
import jax.numpy as jnp
from jax import lax
from jax.experimental import pallas as pl
from jax.experimental.pallas import tpu as pltpu
from jax.experimental.pallas import tpu_sc as plsc

N = 10000
E = 320000
D = 128
NC, NS, L = 2, 16, 16
NW = NC * NS
EPW = E // NW
DEG_R = EPW // L
CH = 50
NCH = EPW // CH
NG = 5
GCH = NCH // NG
RB = 4
ZR = 8
NZCH = N // ZR
WR = 80
NWCH = N // WR

_mesh = plsc.VectorSubcoreMesh(
    core_axis_name="c", subcore_axis_name="s", num_cores=NC, num_subcores=NS
)


@functools.partial(
    pl.kernel,
    out_type=jax.ShapeDtypeStruct((NW, N), jnp.float32),
    mesh=_mesh,
    scratch_types=[
        pltpu.VMEM((DEG_R, L), jnp.int32),
        pltpu.VMEM((N,), jnp.float32),
    ],
    compiler_params=pltpu.CompilerParams(needs_layout_passes=False),
)
def _degree_kernel(dst_hbm, deg_out, dst_v, deg_v):
    wid = lax.axis_index("s") * NC + lax.axis_index("c")
    pltpu.sync_copy(dst_hbm.at[wid], dst_v)

    zeros = jnp.zeros((L,), jnp.float32)

    def zrow(i, carry):
        deg_v[pl.ds(i * L, L)] = zeros
        return carry

    lax.fori_loop(0, N // L, zrow, 0)

    ones = jnp.ones((L,), jnp.float32)

    def body(i, carry):
        idx = dst_v[i, :]
        plsc.addupdate_scatter(deg_v, [idx], ones)
        return carry

    lax.fori_loop(0, DEG_R, body, 0)
    pltpu.sync_copy(deg_v, deg_out.at[wid])


@functools.partial(
    pl.kernel,
    out_type=jax.ShapeDtypeStruct((NC, N, D), jnp.float32),
    mesh=_mesh,
    scratch_types=[
        pltpu.VMEM((GCH, CH), jnp.int32),
        pltpu.VMEM((GCH, CH), jnp.int32),
        pltpu.VMEM((RB, CH, D), jnp.float32),
        pltpu.VMEM((ZR, D), jnp.float32),
        pltpu.VMEM_SHARED((N, D), jnp.float32),
        [pltpu.SemaphoreType.DMA] * RB,
        [pltpu.SemaphoreType.DMA] * RB,
    ],
    compiler_params=pltpu.CompilerParams(needs_layout_passes=False),
)
def _agg_kernel(z_hbm, src_hbm, dst_hbm, agg_out,
                src_v, dst_v, rows_v, zb_v, acc_sh, gsems, ssems):
    cid = lax.axis_index("c")
    sid = lax.axis_index("s")
    wid = sid * NC + cid

    zeros = jnp.zeros((L,), jnp.float32)

    def zbody(k, carry):
        r = k // (D // L)
        c = (k % (D // L)) * L
        zb_v[r, pl.ds(c, L)] = zeros
        return carry

    lax.fori_loop(0, ZR * (D // L), zbody, 0)

    def zstripe(k, carry):
        cidx = sid + NS * k

        @pl.when(cidx < NZCH)
        def _():
            off = pl.multiple_of(cidx * ZR, 8)
            pltpu.sync_copy(zb_v, acc_sh.at[pl.ds(off, ZR)])

        return carry

    lax.fori_loop(0, (NZCH + NS - 1) // NS, zstripe, 0)
    plsc.subcore_barrier()

    def group(g, carry):
        pltpu.sync_copy(src_hbm.at[wid, g], src_v)
        pltpu.sync_copy(dst_hbm.at[wid, g], dst_v)
        for b in range(RB):
            pltpu.async_copy(z_hbm.at[src_v.at[b]], rows_v.at[b], gsems[b])

        def ring(r, carry2):
            j0 = RB * r
            descs = []
            for b in range(RB):
                j = j0 + b
                pltpu.make_async_copy(
                    z_hbm.at[src_v.at[j]], rows_v.at[b], gsems[b]).wait()
                descs.append(pltpu.async_copy(
                    rows_v.at[b], acc_sh.at[dst_v.at[j]], ssems[b], add=True))
            for b in range(RB):
                j = j0 + b
                descs[b].wait()

                @pl.when(j + RB < GCH)
                def _():
                    pltpu.async_copy(
                        z_hbm.at[src_v.at[j + RB]], rows_v.at[b], gsems[b])

            return carry2

        lax.fori_loop(0, GCH // RB, ring, 0)
        return carry

    lax.fori_loop(0, NG, group, 0)

    plsc.subcore_barrier()

    def wstripe(k, carry):
        cidx = sid + NS * k

        @pl.when(cidx < NWCH)
        def _():
            off = pl.multiple_of(cidx * WR, 8)
            pltpu.sync_copy(
                acc_sh.at[pl.ds(off, WR)],
                agg_out.at[cid].at[pl.ds(off, WR)],
            )

        return carry

    lax.fori_loop(0, (NWCH + NS - 1) // NS, wstripe, 0)


BLK = 2000


def _prep_body(x_ref, degp_ref, w0t_ref, b_ref, z_ref, p_ref, norm_ref):
    deg = jnp.sum(degp_ref[...], axis=1, keepdims=True)
    norm = lax.rsqrt(jnp.maximum(deg, 1.0))
    x = x_ref[...]
    z_ref[...] = x * norm
    norm_ref[...] = norm
    p_ref[...] = (
        jnp.dot(x, w0t_ref[...], preferred_element_type=jnp.float32)
        + b_ref[...]
    )


def _prep(x, degp_t, w0t, b2):
    grid = (N // BLK,)
    return pl.pallas_call(
        _prep_body,
        grid=grid,
        in_specs=[
            pl.BlockSpec((BLK, D), lambda i: (i, 0)),
            pl.BlockSpec((BLK, NW), lambda i: (i, 0)),
            pl.BlockSpec((D, D), lambda i: (0, 0)),
            pl.BlockSpec((1, D), lambda i: (0, 0)),
        ],
        out_specs=[
            pl.BlockSpec((BLK, D), lambda i: (i, 0)),
            pl.BlockSpec((BLK, D), lambda i: (i, 0)),
            pl.BlockSpec((BLK, 1), lambda i: (i, 0)),
        ],
        out_shape=[
            jax.ShapeDtypeStruct((N, D), jnp.float32),
            jax.ShapeDtypeStruct((N, D), jnp.float32),
            jax.ShapeDtypeStruct((N, 1), jnp.float32),
        ],
    )(x, degp_t, w0t, b2)


def _final_body(p_ref, agg_ref, norm_ref, w1t_ref, out_ref):
    a = (agg_ref[0] + agg_ref[1]) * norm_ref[...]
    out_ref[...] = p_ref[...] - jnp.dot(
        a, w1t_ref[...], preferred_element_type=jnp.float32
    )


def _final(p, agg, norm, w1t):
    grid = (N // BLK,)
    return pl.pallas_call(
        _final_body,
        grid=grid,
        in_specs=[
            pl.BlockSpec((BLK, D), lambda i: (i, 0)),
            pl.BlockSpec((NC, BLK, D), lambda i: (0, i, 0)),
            pl.BlockSpec((BLK, 1), lambda i: (i, 0)),
            pl.BlockSpec((D, D), lambda i: (0, 0)),
        ],
        out_specs=pl.BlockSpec((BLK, D), lambda i: (i, 0)),
        out_shape=jax.ShapeDtypeStruct((N, D), jnp.float32),
    )(p, agg, norm, w1t)


def kernel(inputs, edge_index, W, b):
    edge_index = edge_index.astype(jnp.int32)
    src2 = edge_index[0].reshape(NW, EPW)
    dst2 = edge_index[1].reshape(NW, EPW)

    degp = _degree_kernel(dst2.reshape(NW, DEG_R, L))
    degp_t = degp.T

    w0t = W[:, :D].T
    w1t = W[:, D:].T
    z, p, norm = _prep(inputs, degp_t, w0t, b.reshape(1, D))

    agg = _agg_kernel(z, src2.reshape(NW, NG, GCH, CH),
                      dst2.reshape(NW, NG, GCH, CH))
    out = _final(p, agg, norm, w1t)
    return out


# speedup vs baseline: 10.7167x; 1.0588x over previous
import functools

import jax
import jax.numpy as jnp
from jax import lax
from jax.experimental import pallas as pl
from jax.experimental.pallas import tpu as pltpu
from jax.experimental.pallas import tpu_sc as plsc

N = 10000
E = 320000
D = 128
NC, NS, L = 2, 16, 16
NW = NC * NS
EPW = E // NW
CH = 50
NCH = EPW // CH
NG = 5
GCH = NCH // NG
RB = 5
ZR = 8
NZCH = N // ZR
WR = 80
NWCH = N // WR

_mesh = plsc.VectorSubcoreMesh(
    core_axis_name="c", subcore_axis_name="s", num_cores=NC, num_subcores=NS
)


@functools.partial(
    pl.kernel,
    out_type=jax.ShapeDtypeStruct((NW, N), jnp.float32),
    mesh=_mesh,
    scratch_types=[
        pltpu.VMEM((GCH, CH), jnp.int32),
        pltpu.VMEM((N,), jnp.float32),
    ],
    compiler_params=pltpu.CompilerParams(needs_layout_passes=False),
)
def _degree_kernel(edge_hbm, deg_out, dst_v, deg_v):
    wid = lax.axis_index("s") * NC + lax.axis_index("c")

    zeros = jnp.zeros((L,), jnp.float32)

    def zrow(i, carry):
        deg_v[pl.ds(i * L, L)] = zeros
        return carry

    lax.fori_loop(0, N // L, zrow, 0)

    ones = jnp.ones((L,), jnp.float32)
    tail_mask = lax.iota(jnp.int32, L) >= ((CH // L) * L - (CH - L))

    def group(g, carry):
        pltpu.sync_copy(edge_hbm.at[1, wid, g], dst_v)

        def body(i, carry2):
            for kk in range(CH // L):
                idx = dst_v[i, pl.ds(kk * L, L)]
                plsc.addupdate_scatter(deg_v, [idx], ones)
            idx = dst_v[i, pl.ds(CH - L, L)]
            plsc.addupdate_scatter(deg_v, [idx], ones, mask=tail_mask)
            return carry2

        lax.fori_loop(0, GCH, body, 0)
        return carry

    lax.fori_loop(0, NG, group, 0)
    pltpu.sync_copy(deg_v, deg_out.at[wid])


@functools.partial(
    pl.kernel,
    out_type=jax.ShapeDtypeStruct((NC, N, D), jnp.float32),
    mesh=_mesh,
    scratch_types=[
        pltpu.VMEM((GCH, CH), jnp.int32),
        pltpu.VMEM((GCH, CH), jnp.int32),
        pltpu.VMEM((RB, CH, D), jnp.float32),
        pltpu.VMEM((ZR, D), jnp.float32),
        pltpu.VMEM_SHARED((N, D), jnp.float32),
        [pltpu.SemaphoreType.DMA] * RB,
        [pltpu.SemaphoreType.DMA] * RB,
    ],
    compiler_params=pltpu.CompilerParams(needs_layout_passes=False),
)
def _agg_kernel(z_hbm, edge_hbm, agg_out,
                src_v, dst_v, rows_v, zb_v, acc_sh, gsems, ssems):
    cid = lax.axis_index("c")
    sid = lax.axis_index("s")
    wid = sid * NC + cid

    zeros = jnp.zeros((L,), jnp.float32)

    def zbody(k, carry):
        r = k // (D // L)
        c = (k % (D // L)) * L
        zb_v[r, pl.ds(c, L)] = zeros
        return carry

    lax.fori_loop(0, ZR * (D // L), zbody, 0)

    def zstripe(k, carry):
        cidx = sid + NS * k

        @pl.when(cidx < NZCH)
        def _():
            off = pl.multiple_of(cidx * ZR, 8)
            pltpu.sync_copy(zb_v, acc_sh.at[pl.ds(off, ZR)])

        return carry

    lax.fori_loop(0, (NZCH + NS - 1) // NS, zstripe, 0)
    plsc.subcore_barrier()

    def group(g, carry):
        pltpu.sync_copy(edge_hbm.at[0, wid, g], src_v)
        pltpu.sync_copy(edge_hbm.at[1, wid, g], dst_v)
        for b in range(RB):
            pltpu.async_copy(z_hbm.at[src_v.at[b]], rows_v.at[b], gsems[b])

        def ring(r, carry2):
            j0 = RB * r
            descs = []
            for b in range(RB):
                j = j0 + b
                pltpu.make_async_copy(
                    z_hbm.at[src_v.at[j]], rows_v.at[b], gsems[b]).wait()
                descs.append(pltpu.async_copy(
                    rows_v.at[b], acc_sh.at[dst_v.at[j]], ssems[b], add=True))
            for b in range(RB):
                j = j0 + b
                descs[b].wait()

                @pl.when(j + RB < GCH)
                def _():
                    pltpu.async_copy(
                        z_hbm.at[src_v.at[j + RB]], rows_v.at[b], gsems[b])

            return carry2

        lax.fori_loop(0, GCH // RB, ring, 0)
        return carry

    lax.fori_loop(0, NG, group, 0)

    plsc.subcore_barrier()

    def wstripe(k, carry):
        cidx = sid + NS * k

        @pl.when(cidx < NWCH)
        def _():
            off = pl.multiple_of(cidx * WR, 8)
            pltpu.sync_copy(
                acc_sh.at[pl.ds(off, WR)],
                agg_out.at[cid].at[pl.ds(off, WR)],
            )

        return carry

    lax.fori_loop(0, (NWCH + NS - 1) // NS, wstripe, 0)


BLK = 2000


def _prep_body(x_ref, degp_ref, z_ref, norm_ref):
    deg = jnp.sum(degp_ref[...], axis=1, keepdims=True)
    norm = lax.rsqrt(jnp.maximum(deg, 1.0))
    z_ref[...] = x_ref[...] * norm
    norm_ref[...] = norm


def _prep(x, degp_t):
    grid = (N // BLK,)
    return pl.pallas_call(
        _prep_body,
        grid=grid,
        in_specs=[
            pl.BlockSpec((BLK, D), lambda i: (i, 0)),
            pl.BlockSpec((BLK, NW), lambda i: (i, 0)),
        ],
        out_specs=[
            pl.BlockSpec((BLK, D), lambda i: (i, 0)),
            pl.BlockSpec((BLK, 1), lambda i: (i, 0)),
        ],
        out_shape=[
            jax.ShapeDtypeStruct((N, D), jnp.float32),
            jax.ShapeDtypeStruct((N, 1), jnp.float32),
        ],
    )(x, degp_t)


def _final_body(x_ref, agg_ref, norm_ref, w0t_ref, w1t_ref, b_ref, out_ref):
    a = (agg_ref[0] + agg_ref[1]) * norm_ref[...]
    p = jnp.dot(x_ref[...], w0t_ref[...], preferred_element_type=jnp.float32)
    out_ref[...] = p + b_ref[...] - jnp.dot(
        a, w1t_ref[...], preferred_element_type=jnp.float32
    )


def _final(x, agg, norm, w0t, w1t, b2):
    grid = (N // BLK,)
    return pl.pallas_call(
        _final_body,
        grid=grid,
        in_specs=[
            pl.BlockSpec((BLK, D), lambda i: (i, 0)),
            pl.BlockSpec((NC, BLK, D), lambda i: (0, i, 0)),
            pl.BlockSpec((BLK, 1), lambda i: (i, 0)),
            pl.BlockSpec((D, D), lambda i: (0, 0)),
            pl.BlockSpec((D, D), lambda i: (0, 0)),
            pl.BlockSpec((1, D), lambda i: (0, 0)),
        ],
        out_specs=pl.BlockSpec((BLK, D), lambda i: (i, 0)),
        out_shape=jax.ShapeDtypeStruct((N, D), jnp.float32),
    )(x, agg, norm, w0t, w1t, b2)


def kernel(inputs, edge_index, W, b):
    edge5 = edge_index.astype(jnp.int32).reshape(2, NW, NG, GCH, CH)

    degp = _degree_kernel(edge5)
    degp_t = degp.T

    w0t = W[:, :D].T
    w1t = W[:, D:].T
    z, norm = _prep(inputs, degp_t)

    agg = _agg_kernel(z, edge5)
    out = _final(inputs, agg, norm, w0t, w1t, b.reshape(1, D))
    return out
